# baseline (speedup 1.0000x reference)
import functools

import jax
import jax.numpy as jnp
from jax import lax
from jax.experimental import pallas as pl
from jax.experimental.pallas import tpu as pltpu
from jax.experimental.pallas import tpu_sc as plsc

TM = 128


def _router_body(x_ref, wg_ref, bg_ref, emat_ref, counts_ref,
                 wrep0_ref, wrep1_ref, carry_ref):
    step = pl.program_id(0)

    @pl.when(step == 0)
    def _():
        carry_ref[...] = jnp.zeros_like(carry_ref)

    x = x_ref[...].astype(jnp.bfloat16)
    logits = jnp.dot(x, wg_ref[...].astype(jnp.bfloat16),
                     preferred_element_type=jnp.float32)
    logits = logits + bg_ref[...]
    rt, lanes = logits.shape
    lane = jax.lax.broadcasted_iota(jnp.int32, (rt, lanes), 1)
    m0 = jnp.max(logits, axis=1, keepdims=True)
    i0 = jnp.min(jnp.where(logits == m0, lane, lanes), axis=1, keepdims=True)
    masked = jnp.where(lane == i0, -jnp.inf, logits)
    m1 = jnp.max(masked, axis=1, keepdims=True)
    i1 = jnp.min(jnp.where(masked == m1, lane, lanes), axis=1, keepdims=True)
    s = jnp.exp(m1 - m0)
    w0 = 1.0 / (1.0 + s)
    w1 = 1.0 - w0

    oh0 = (lane == i0).astype(jnp.float32)
    oh1 = (lane == i1).astype(jnp.float32)
    ohsum = oh0 + oh1
    r_i = jax.lax.broadcasted_iota(jnp.int32, (rt, rt), 0)
    r_j = jax.lax.broadcasted_iota(jnp.int32, (rt, rt), 1)
    ltri = (r_i > r_j).astype(jnp.float32)
    pref = jnp.dot(ltri, ohsum, preferred_element_type=jnp.float32)
    pref = pref + carry_ref[...]
    carry_ref[...] += jnp.sum(ohsum, axis=0, keepdims=True)
    rank0 = jnp.sum(oh0 * pref, axis=1, keepdims=True).astype(jnp.int32)
    rank1 = jnp.sum(oh1 * pref, axis=1, keepdims=True).astype(jnp.int32)

    emat_ref[...] = jnp.where(
        lane == 0, i0, jnp.where(
            lane == 1, i1, jnp.where(
                lane == 2, rank0, jnp.where(lane == 3, rank1, 0))))
    counts_ref[...] = carry_ref[...].astype(jnp.int32)
    wrep0_ref[...] = jnp.broadcast_to(w0, wrep0_ref.shape)
    wrep1_ref[...] = jnp.broadcast_to(w1, wrep1_ref.shape)


def _router(xf, Wg, bg, *, rt=1024):
    t, d = xf.shape
    e = Wg.shape[1]
    lanes = 128
    wg_pad = jnp.zeros((d, lanes), jnp.float32).at[:, :e].set(Wg)
    bg_pad = jnp.full((1, lanes), -jnp.inf, jnp.float32).at[0, :e].set(bg)
    emat, counts, wrep0, wrep1 = pl.pallas_call(
        _router_body,
        grid=(t // rt,),
        in_specs=[
            pl.BlockSpec((rt, d), lambda i: (i, 0)),
            pl.BlockSpec((d, lanes), lambda i: (0, 0)),
            pl.BlockSpec((1, lanes), lambda i: (0, 0)),
        ],
        out_specs=[
            pl.BlockSpec((rt, lanes), lambda i: (i, 0)),
            pl.BlockSpec((1, lanes), lambda i: (0, 0)),
            pl.BlockSpec((rt, 128), lambda i: (i, 0)),
            pl.BlockSpec((rt, 128), lambda i: (i, 0)),
        ],
        out_shape=[
            jax.ShapeDtypeStruct((t, lanes), jnp.int32),
            jax.ShapeDtypeStruct((1, lanes), jnp.int32),
            jax.ShapeDtypeStruct((t, 128), jnp.float32),
            jax.ShapeDtypeStruct((t, 128), jnp.float32),
        ],
        scratch_shapes=[pltpu.VMEM((1, lanes), jnp.float32)],
    )(xf, wg_pad, bg_pad)
    return emat, counts, wrep0, wrep1


def _sc_dispatch(xf, slot0, slot1, wrep0, wrep1, np_rows):
    t, d = xf.shape
    info = plsc.get_sparse_core_info()
    nw = info.num_cores * info.num_subcores
    per_w = t // nw
    chunk = 32
    nch = per_w // chunk
    mesh = plsc.VectorSubcoreMesh(core_axis_name="c", subcore_axis_name="s")

    @functools.partial(
        pl.kernel, mesh=mesh,
        out_type=[
            jax.ShapeDtypeStruct((np_rows, d), jnp.float32),
            jax.ShapeDtypeStruct((np_rows, 128), jnp.float32),
        ],
        scratch_types=[
            pltpu.VMEM((2 * nch, chunk), jnp.int32),
            pltpu.VMEM((chunk, d), jnp.float32),
            pltpu.VMEM((chunk, d), jnp.float32),
            pltpu.VMEM((per_w, 128), jnp.float32),
            pltpu.VMEM((per_w, 128), jnp.float32),
            pltpu.SemaphoreType.DMA,
            pltpu.SemaphoreType.DMA,
            pltpu.SemaphoreType.DMA,
            pltpu.SemaphoreType.DMA,
            pltpu.SemaphoreType.DMA,
        ],
    )
    def dispatch_k(x_hbm, s0_hbm, s1_hbm, w0_hbm, w1_hbm,
                   xs_hbm, wr_hbm,
                   idx_v, rows0, rows1, w0_v, w1_v,
                   ld0, ld1, st0, st1, stw):
        wid = lax.axis_index("s") * info.num_cores + lax.axis_index("c")
        base = wid * per_w
        psl = pl.ds(base, per_w)
        pltpu.sync_copy(w0_hbm.at[psl], w0_v)
        pltpu.sync_copy(w1_hbm.at[psl], w1_v)
        for c in range(nch):
            csl2 = pl.ds(base + c * chunk, chunk)
            pltpu.sync_copy(s0_hbm.at[csl2], idx_v.at[c])
            pltpu.sync_copy(s1_hbm.at[csl2], idx_v.at[nch + c])

        rows = (rows0, rows1)
        lsem = (ld0, ld1)
        lcp = {}

        def fire(c):
            bi = c % 2
            lcp[c] = pltpu.async_copy(
                x_hbm.at[pl.ds(base + c * chunk, chunk)], rows[bi], lsem[bi])

        fire(0)
        scp = {}
        for c in range(nch):
            bi = c % 2
            lcp[c].wait()
            csl = pl.ds(c * chunk, chunk)
            scp[(c, 0)] = pltpu.async_copy(
                rows[bi], xs_hbm.at[idx_v.at[c]], st0)
            scp[(c, 1)] = pltpu.async_copy(
                rows[bi], xs_hbm.at[idx_v.at[nch + c]], st1)
            scp[(c, 2)] = pltpu.async_copy(
                w0_v.at[csl], wr_hbm.at[idx_v.at[c]], stw)
            scp[(c, 3)] = pltpu.async_copy(
                w1_v.at[csl], wr_hbm.at[idx_v.at[nch + c]], stw)
            if c + 1 < nch:
                if c >= 1:
                    for q in range(4):
                        scp[(c - 1, q)].wait()
                fire(c + 1)
        for c in (nch - 2, nch - 1):
            for q in range(4):
                scp[(c, q)].wait()

    return dispatch_k(xf, slot0, slot1, wrep0, wrep1)


def _ffn_body(te_ref, xs_ref, w1_ref, w2_ref, b1_ref, b2_ref, wrow_ref,
              out_ref):
    pre = jnp.dot(xs_ref[...], w1_ref[0], preferred_element_type=jnp.float32)
    pre = pre + b1_ref[0]
    h = (0.5 * pre * (1.0 + jax.lax.erf(pre * 0.7071067811865476))
         ).astype(jnp.bfloat16)
    y = jnp.dot(h, w2_ref[0], preferred_element_type=jnp.float32)
    out_ref[...] = (y + b2_ref[0]) * wrow_ref[:, 0:1]


def _grouped_ffn(xs, W1, b1, W2, b2, tile_e, wrow):
    np_rows, d = xs.shape
    ne, _, dff = W1.shape
    nt = np_rows // TM
    b1r = b1.reshape(ne, 1, dff)
    b2r = b2.reshape(ne, 1, d)
    return pl.pallas_call(
        _ffn_body,
        grid_spec=pltpu.PrefetchScalarGridSpec(
            num_scalar_prefetch=1,
            grid=(nt,),
            in_specs=[
                pl.BlockSpec((TM, d), lambda i, te: (i, 0)),
                pl.BlockSpec((1, d, dff), lambda i, te: (te[i], 0, 0)),
                pl.BlockSpec((1, dff, d), lambda i, te: (te[i], 0, 0),
                             pipeline_mode=pl.Buffered(buffer_count=1)),
                pl.BlockSpec((1, 1, dff), lambda i, te: (te[i], 0, 0)),
                pl.BlockSpec((1, 1, d), lambda i, te: (te[i], 0, 0)),
                pl.BlockSpec((TM, 128), lambda i, te: (i, 0)),
            ],
            out_specs=pl.BlockSpec((TM, d), lambda i, te: (i, 0)),
        ),
        out_shape=jax.ShapeDtypeStruct((np_rows, d), jnp.float32),
    )(tile_e, xs, W1, W2, b1r, b2r, wrow)


def _sc_combine(yw, slot0, slot1):
    np_rows, d = yw.shape
    t = slot0.shape[0]
    info = plsc.get_sparse_core_info()
    nw = info.num_cores * info.num_subcores
    per_w = t // nw
    chunk = 16
    nch = per_w // chunk
    nvec = d // 16
    mesh = plsc.VectorSubcoreMesh(core_axis_name="c", subcore_axis_name="s")

    @functools.partial(
        pl.kernel, mesh=mesh,
        out_type=jax.ShapeDtypeStruct((t, d), jnp.float32),
        scratch_types=[
            pltpu.VMEM((per_w,), jnp.int32),
            pltpu.VMEM((per_w,), jnp.int32),
            pltpu.VMEM((chunk, d), jnp.float32),
            pltpu.VMEM((chunk, d), jnp.float32),
            pltpu.VMEM((chunk, d), jnp.float32),
            pltpu.VMEM((chunk, d), jnp.float32),
            pltpu.SemaphoreType.DMA,
            pltpu.SemaphoreType.DMA,
            pltpu.SemaphoreType.DMA,
            pltpu.SemaphoreType.DMA,
            pltpu.SemaphoreType.DMA,
            pltpu.SemaphoreType.DMA,
        ],
    )
    def combine_k(y_hbm, s0_hbm, s1_hbm, out_hbm,
                  i0_v, i1_v,
                  a0, a1, b0, b1, ga0, ga1, gb0, gb1, oa0, oa1):
        wid = lax.axis_index("s") * info.num_cores + lax.axis_index("c")
        base = wid * per_w
        pltpu.sync_copy(s0_hbm.at[pl.ds(base, per_w)], i0_v)
        pltpu.sync_copy(s1_hbm.at[pl.ds(base, per_w)], i1_v)
        bufa = (a0, a1)
        bufb = (b0, b1)
        gsa = (ga0, ga1)
        gsb = (gb0, gb1)
        osa = (oa0, oa1)
        cpa = {}
        cpb = {}
        ocp = {}

        def fire(j):
            bi = j % 2
            sl = pl.ds(j * chunk, chunk)
            cpa[j] = pltpu.async_copy(y_hbm.at[i0_v.at[sl]], bufa[bi], gsa[bi])
            cpb[j] = pltpu.async_copy(y_hbm.at[i1_v.at[sl]], bufb[bi], gsb[bi])

        fire(0)
        for j in range(nch):
            bi = j % 2
            if j + 1 < nch:
                if j + 1 >= 2:
                    ocp[j - 1].wait()
                fire(j + 1)
            cpa[j].wait()
            cpb[j].wait()
            a = bufa[bi]
            b = bufb[bi]
            for r in range(chunk):
                def add_body(c, carry2, a=a, b=b, r=r):
                    sl2 = pl.ds(c * 16, 16)
                    a[r, sl2] = a[r, sl2] + b[r, sl2]
                    return carry2

                lax.fori_loop(0, nvec, add_body, 0)
            ocp[j] = pltpu.async_copy(
                bufa[bi], out_hbm.at[pl.ds(base + j * chunk, chunk)], osa[bi])
        ocp[nch - 2].wait()
        ocp[nch - 1].wait()

    return combine_k(yw, slot0, slot1)


def kernel(x, Wg, bg, W1, b1, W2, b2):
    b, s, d = x.shape
    ne = W1.shape[0]
    t = b * s
    xf = x.reshape(t, d)
    emat, counts, wrep0, wrep1 = _router(xf, Wg, bg)

    counts8 = counts[0, :ne]
    padded = ((counts8 + TM - 1) // TM) * TM
    off_full = jnp.concatenate([jnp.zeros(1, jnp.int32), jnp.cumsum(padded)])
    off = off_full[:ne]
    np_rows = 2 * t + ne * TM
    nt = np_rows // TM
    starts = jnp.arange(nt, dtype=jnp.int32) * TM
    tile_e = jnp.minimum(
        jnp.sum((starts[:, None] >= off_full[None, 1:]).astype(jnp.int32),
                axis=1), ne - 1)

    e0a = emat[:, 0]
    e1a = emat[:, 1]
    r0a = emat[:, 2]
    r1a = emat[:, 3]

    slot0 = r0a
    slot1 = r1a
    for k in range(ne):
        slot0 = slot0 + jnp.where(e0a == k, off[k], 0)
        slot1 = slot1 + jnp.where(e1a == k, off[k], 0)

    xs, wrow = _sc_dispatch(xf, slot0, slot1, wrep0, wrep1, np_rows)
    yw = _grouped_ffn(xs, W1, b1, W2, b2, tile_e, wrow)
    outf = _sc_combine(yw, slot0, slot1)
    return outf.reshape(b, s, d)


# speedup vs baseline: 1.0014x; 1.0014x over previous
import functools

import jax
import jax.numpy as jnp
from jax import lax
from jax.experimental import pallas as pl
from jax.experimental.pallas import tpu as pltpu
from jax.experimental.pallas import tpu_sc as plsc

TM = 128


def _router_body(x_ref, wg_ref, bg_ref, emat_ref, counts_ref,
                 wrep0_ref, wrep1_ref, carry_ref):
    step = pl.program_id(0)

    @pl.when(step == 0)
    def _():
        carry_ref[...] = jnp.zeros_like(carry_ref)

    x = x_ref[...].astype(jnp.bfloat16)
    logits = jnp.dot(x, wg_ref[...].astype(jnp.bfloat16),
                     preferred_element_type=jnp.float32)
    logits = logits + bg_ref[...]
    rt, lanes = logits.shape
    lane = jax.lax.broadcasted_iota(jnp.int32, (rt, lanes), 1)
    m0 = jnp.max(logits, axis=1, keepdims=True)
    i0 = jnp.min(jnp.where(logits == m0, lane, lanes), axis=1, keepdims=True)
    masked = jnp.where(lane == i0, -jnp.inf, logits)
    m1 = jnp.max(masked, axis=1, keepdims=True)
    i1 = jnp.min(jnp.where(masked == m1, lane, lanes), axis=1, keepdims=True)
    s = jnp.exp(m1 - m0)
    w0 = 1.0 / (1.0 + s)
    w1 = 1.0 - w0

    oh0 = (lane == i0).astype(jnp.float32)
    oh1 = (lane == i1).astype(jnp.float32)
    ohsum = oh0 + oh1
    r_i = jax.lax.broadcasted_iota(jnp.int32, (rt, rt), 0)
    r_j = jax.lax.broadcasted_iota(jnp.int32, (rt, rt), 1)
    ltri = (r_i > r_j).astype(jnp.float32)
    pref = jnp.dot(ltri, ohsum, preferred_element_type=jnp.float32)
    pref = pref + carry_ref[...]
    carry_ref[...] += jnp.sum(ohsum, axis=0, keepdims=True)
    rank0 = jnp.sum(oh0 * pref, axis=1, keepdims=True).astype(jnp.int32)
    rank1 = jnp.sum(oh1 * pref, axis=1, keepdims=True).astype(jnp.int32)

    emat_ref[...] = jnp.where(
        lane == 0, i0, jnp.where(
            lane == 1, i1, jnp.where(
                lane == 2, rank0, jnp.where(lane == 3, rank1, 0))))
    counts_ref[...] = carry_ref[...].astype(jnp.int32)
    wrep0_ref[...] = jnp.broadcast_to(w0, wrep0_ref.shape)
    wrep1_ref[...] = jnp.broadcast_to(w1, wrep1_ref.shape)


def _router(xf, Wg, bg, *, rt=1024):
    t, d = xf.shape
    e = Wg.shape[1]
    lanes = 128
    wg_pad = jnp.zeros((d, lanes), jnp.float32).at[:, :e].set(Wg)
    bg_pad = jnp.full((1, lanes), -jnp.inf, jnp.float32).at[0, :e].set(bg)
    emat, counts, wrep0, wrep1 = pl.pallas_call(
        _router_body,
        grid=(t // rt,),
        in_specs=[
            pl.BlockSpec((rt, d), lambda i: (i, 0)),
            pl.BlockSpec((d, lanes), lambda i: (0, 0)),
            pl.BlockSpec((1, lanes), lambda i: (0, 0)),
        ],
        out_specs=[
            pl.BlockSpec((rt, lanes), lambda i: (i, 0)),
            pl.BlockSpec((1, lanes), lambda i: (0, 0)),
            pl.BlockSpec((rt, 128), lambda i: (i, 0)),
            pl.BlockSpec((rt, 128), lambda i: (i, 0)),
        ],
        out_shape=[
            jax.ShapeDtypeStruct((t, lanes), jnp.int32),
            jax.ShapeDtypeStruct((1, lanes), jnp.int32),
            jax.ShapeDtypeStruct((t, 128), jnp.float32),
            jax.ShapeDtypeStruct((t, 128), jnp.float32),
        ],
        scratch_shapes=[pltpu.VMEM((1, lanes), jnp.float32)],
    )(xf, wg_pad, bg_pad)
    return emat, counts, wrep0, wrep1


def _sc_dispatch(xf, slot0, slot1, wrep0, wrep1, np_rows):
    t, d = xf.shape
    info = plsc.get_sparse_core_info()
    nw = info.num_cores * info.num_subcores
    per_w = t // nw
    chunk = 32
    nch = per_w // chunk
    mesh = plsc.VectorSubcoreMesh(core_axis_name="c", subcore_axis_name="s")

    @functools.partial(
        pl.kernel, mesh=mesh,
        out_type=[
            jax.ShapeDtypeStruct((np_rows, d), jnp.float32),
            jax.ShapeDtypeStruct((np_rows, 128), jnp.float32),
        ],
        scratch_types=[
            pltpu.VMEM((2 * nch, chunk), jnp.int32),
            pltpu.VMEM((chunk, d), jnp.float32),
            pltpu.VMEM((chunk, d), jnp.float32),
            pltpu.VMEM((per_w, 128), jnp.float32),
            pltpu.VMEM((per_w, 128), jnp.float32),
            pltpu.SemaphoreType.DMA,
            pltpu.SemaphoreType.DMA,
            pltpu.SemaphoreType.DMA,
            pltpu.SemaphoreType.DMA,
            pltpu.SemaphoreType.DMA,
        ],
    )
    def dispatch_k(x_hbm, s0_hbm, s1_hbm, w0_hbm, w1_hbm,
                   xs_hbm, wr_hbm,
                   idx_v, rows0, rows1, w0_v, w1_v,
                   ld0, ld1, st0, st1, stw):
        wid = lax.axis_index("s") * info.num_cores + lax.axis_index("c")
        base = wid * per_w
        psl = pl.ds(base, per_w)
        pltpu.sync_copy(w0_hbm.at[psl], w0_v)
        pltpu.sync_copy(w1_hbm.at[psl], w1_v)
        for c in range(nch):
            csl2 = pl.ds(base + c * chunk, chunk)
            pltpu.sync_copy(s0_hbm.at[csl2], idx_v.at[c])
            pltpu.sync_copy(s1_hbm.at[csl2], idx_v.at[nch + c])

        rows = (rows0, rows1)
        lsem = (ld0, ld1)
        lcp = {}

        def fire(c):
            bi = c % 2
            lcp[c] = pltpu.async_copy(
                x_hbm.at[pl.ds(base + c * chunk, chunk)], rows[bi], lsem[bi])

        fire(0)
        scp = {}
        for c in range(nch):
            bi = c % 2
            lcp[c].wait()
            csl = pl.ds(c * chunk, chunk)
            scp[(c, 0)] = pltpu.async_copy(
                rows[bi], xs_hbm.at[idx_v.at[c]], st0)
            scp[(c, 1)] = pltpu.async_copy(
                rows[bi], xs_hbm.at[idx_v.at[nch + c]], st1)
            scp[(c, 2)] = pltpu.async_copy(
                w0_v.at[csl], wr_hbm.at[idx_v.at[c]], stw)
            scp[(c, 3)] = pltpu.async_copy(
                w1_v.at[csl], wr_hbm.at[idx_v.at[nch + c]], stw)
            if c + 1 < nch:
                if c >= 1:
                    for q in range(4):
                        scp[(c - 1, q)].wait()
                fire(c + 1)
        for c in (nch - 2, nch - 1):
            for q in range(4):
                scp[(c, q)].wait()

    return dispatch_k(xf, slot0, slot1, wrep0, wrep1)


def _ffn_body(te_ref, xs_ref, w1_ref, w2_ref, b1_ref, b2_ref, wrow_ref,
              out_ref):
    pre = jnp.dot(xs_ref[...], w1_ref[0], preferred_element_type=jnp.float32)
    pre = pre + b1_ref[0]
    h = (0.5 * pre * (1.0 + jax.lax.erf(pre * 0.7071067811865476))
         ).astype(jnp.bfloat16)
    y = jnp.dot(h, w2_ref[0], preferred_element_type=jnp.float32)
    out_ref[...] = (y + b2_ref[0]) * wrow_ref[:, 0:1]


def _grouped_ffn(xs, W1, b1, W2, b2, tile_e, wrow):
    np_rows, d = xs.shape
    ne, _, dff = W1.shape
    nt = np_rows // TM
    b1r = b1.reshape(ne, 1, dff)
    b2r = b2.reshape(ne, 1, d)
    return pl.pallas_call(
        _ffn_body,
        grid_spec=pltpu.PrefetchScalarGridSpec(
            num_scalar_prefetch=1,
            grid=(nt,),
            in_specs=[
                pl.BlockSpec((TM, d), lambda i, te: (i, 0)),
                pl.BlockSpec((1, d, dff), lambda i, te: (te[i], 0, 0)),
                pl.BlockSpec((1, dff, d), lambda i, te: (te[i], 0, 0),
                             pipeline_mode=pl.Buffered(buffer_count=1)),
                pl.BlockSpec((1, 1, dff), lambda i, te: (te[i], 0, 0)),
                pl.BlockSpec((1, 1, d), lambda i, te: (te[i], 0, 0)),
                pl.BlockSpec((TM, 128), lambda i, te: (i, 0)),
            ],
            out_specs=pl.BlockSpec((TM, d), lambda i, te: (i, 0)),
        ),
        out_shape=jax.ShapeDtypeStruct((np_rows, d), jnp.float32),
    )(tile_e, xs, W1, W2, b1r, b2r, wrow)


def _sc_combine(yw, slot0, slot1):
    np_rows, d = yw.shape
    t = slot0.shape[0]
    info = plsc.get_sparse_core_info()
    nw = info.num_cores * info.num_subcores
    per_w = t // nw
    chunk = 16
    nch = per_w // chunk
    nvec = d // 16
    mesh = plsc.VectorSubcoreMesh(core_axis_name="c", subcore_axis_name="s")

    @functools.partial(
        pl.kernel, mesh=mesh,
        out_type=jax.ShapeDtypeStruct((t, d), jnp.float32),
        scratch_types=[
            pltpu.VMEM((per_w,), jnp.int32),
            pltpu.VMEM((per_w,), jnp.int32),
            pltpu.VMEM((chunk, d), jnp.float32),
            pltpu.VMEM((chunk, d), jnp.float32),
            pltpu.VMEM((chunk, d), jnp.float32),
            pltpu.VMEM((chunk, d), jnp.float32),
            pltpu.VMEM((chunk, d), jnp.float32),
            pltpu.VMEM((chunk, d), jnp.float32),
            pltpu.SemaphoreType.DMA,
            pltpu.SemaphoreType.DMA,
            pltpu.SemaphoreType.DMA,
            pltpu.SemaphoreType.DMA,
            pltpu.SemaphoreType.DMA,
            pltpu.SemaphoreType.DMA,
            pltpu.SemaphoreType.DMA,
            pltpu.SemaphoreType.DMA,
            pltpu.SemaphoreType.DMA,
        ],
    )
    def combine_k(y_hbm, s0_hbm, s1_hbm, out_hbm,
                  i0_v, i1_v,
                  a0, a1, a2, b0, b1, b2,
                  ga0, ga1, ga2, gb0, gb1, gb2, oa0, oa1, oa2):
        wid = lax.axis_index("s") * info.num_cores + lax.axis_index("c")
        base = wid * per_w
        pltpu.sync_copy(s0_hbm.at[pl.ds(base, per_w)], i0_v)
        pltpu.sync_copy(s1_hbm.at[pl.ds(base, per_w)], i1_v)
        bufa = (a0, a1, a2)
        bufb = (b0, b1, b2)
        gsa = (ga0, ga1, ga2)
        gsb = (gb0, gb1, gb2)
        osa = (oa0, oa1, oa2)
        cpa = {}
        cpb = {}
        ocp = {}

        def fire(j):
            bi = j % 3
            sl = pl.ds(j * chunk, chunk)
            cpa[j] = pltpu.async_copy(y_hbm.at[i0_v.at[sl]], bufa[bi], gsa[bi])
            cpb[j] = pltpu.async_copy(y_hbm.at[i1_v.at[sl]], bufb[bi], gsb[bi])

        fire(0)
        fire(1)
        for j in range(nch):
            bi = j % 3
            if j + 2 < nch:
                if j + 2 >= 3:
                    ocp[j - 1].wait()
                fire(j + 2)
            cpa[j].wait()
            cpb[j].wait()
            a = bufa[bi]
            b = bufb[bi]
            for r in range(chunk):
                def add_body(c, carry2, a=a, b=b, r=r):
                    sl2 = pl.ds(c * 16, 16)
                    a[r, sl2] = a[r, sl2] + b[r, sl2]
                    return carry2

                lax.fori_loop(0, nvec, add_body, 0)
            ocp[j] = pltpu.async_copy(
                bufa[bi], out_hbm.at[pl.ds(base + j * chunk, chunk)], osa[bi])
        ocp[nch - 3].wait()
        ocp[nch - 2].wait()
        ocp[nch - 1].wait()

    return combine_k(yw, slot0, slot1)


def kernel(x, Wg, bg, W1, b1, W2, b2):
    b, s, d = x.shape
    ne = W1.shape[0]
    t = b * s
    xf = x.reshape(t, d)
    emat, counts, wrep0, wrep1 = _router(xf, Wg, bg)

    counts8 = counts[0, :ne]
    padded = ((counts8 + TM - 1) // TM) * TM
    off_full = jnp.concatenate([jnp.zeros(1, jnp.int32), jnp.cumsum(padded)])
    off = off_full[:ne]
    np_rows = 2 * t + ne * TM
    nt = np_rows // TM
    starts = jnp.arange(nt, dtype=jnp.int32) * TM
    tile_e = jnp.minimum(
        jnp.sum((starts[:, None] >= off_full[None, 1:]).astype(jnp.int32),
                axis=1), ne - 1)

    e0a = emat[:, 0]
    e1a = emat[:, 1]
    r0a = emat[:, 2]
    r1a = emat[:, 3]

    slot0 = r0a
    slot1 = r1a
    for k in range(ne):
        slot0 = slot0 + jnp.where(e0a == k, off[k], 0)
        slot1 = slot1 + jnp.where(e1a == k, off[k], 0)

    xs, wrow = _sc_dispatch(xf, slot0, slot1, wrep0, wrep1, np_rows)
    yw = _grouped_ffn(xs, W1, b1, W2, b2, tile_e, wrow)
    outf = _sc_combine(yw, slot0, slot1)
    return outf.reshape(b, s, d)


# speedup vs baseline: 1.0465x; 1.0450x over previous
import functools

import jax
import jax.numpy as jnp
from jax import lax
from jax.experimental import pallas as pl
from jax.experimental.pallas import tpu as pltpu
from jax.experimental.pallas import tpu_sc as plsc

TM = 128


def _router_body(x_ref, wg_ref, bg_ref, emat_ref, counts_ref,
                 wrep0_ref, wrep1_ref, carry_ref):
    step = pl.program_id(0)

    @pl.when(step == 0)
    def _():
        carry_ref[...] = jnp.zeros_like(carry_ref)

    x = x_ref[...].astype(jnp.bfloat16)
    logits = jnp.dot(x, wg_ref[...].astype(jnp.bfloat16),
                     preferred_element_type=jnp.float32)
    logits = logits + bg_ref[...]
    rt, lanes = logits.shape
    lane = jax.lax.broadcasted_iota(jnp.int32, (rt, lanes), 1)
    m0 = jnp.max(logits, axis=1, keepdims=True)
    i0 = jnp.min(jnp.where(logits == m0, lane, lanes), axis=1, keepdims=True)
    masked = jnp.where(lane == i0, -jnp.inf, logits)
    m1 = jnp.max(masked, axis=1, keepdims=True)
    i1 = jnp.min(jnp.where(masked == m1, lane, lanes), axis=1, keepdims=True)
    s = jnp.exp(m1 - m0)
    w0 = 1.0 / (1.0 + s)
    w1 = 1.0 - w0

    oh0 = (lane == i0).astype(jnp.float32)
    oh1 = (lane == i1).astype(jnp.float32)
    ohsum = oh0 + oh1
    r_i = jax.lax.broadcasted_iota(jnp.int32, (rt, rt), 0)
    r_j = jax.lax.broadcasted_iota(jnp.int32, (rt, rt), 1)
    ltri = (r_i > r_j).astype(jnp.float32)
    pref = jnp.dot(ltri, ohsum, preferred_element_type=jnp.float32)
    pref = pref + carry_ref[...]
    carry_ref[...] += jnp.sum(ohsum, axis=0, keepdims=True)
    rank0 = jnp.sum(oh0 * pref, axis=1, keepdims=True).astype(jnp.int32)
    rank1 = jnp.sum(oh1 * pref, axis=1, keepdims=True).astype(jnp.int32)

    emat_ref[...] = jnp.where(
        lane == 0, i0, jnp.where(
            lane == 1, i1, jnp.where(
                lane == 2, rank0, jnp.where(lane == 3, rank1, 0))))
    counts_ref[...] = carry_ref[...].astype(jnp.int32)
    wrep0_ref[...] = jnp.broadcast_to(w0, wrep0_ref.shape)
    wrep1_ref[...] = jnp.broadcast_to(w1, wrep1_ref.shape)


def _router(xf, Wg, bg, *, rt=1024):
    t, d = xf.shape
    e = Wg.shape[1]
    lanes = 128
    wg_pad = jnp.zeros((d, lanes), jnp.float32).at[:, :e].set(Wg)
    bg_pad = jnp.full((1, lanes), -jnp.inf, jnp.float32).at[0, :e].set(bg)
    emat, counts, wrep0, wrep1 = pl.pallas_call(
        _router_body,
        grid=(t // rt,),
        in_specs=[
            pl.BlockSpec((rt, d), lambda i: (i, 0)),
            pl.BlockSpec((d, lanes), lambda i: (0, 0)),
            pl.BlockSpec((1, lanes), lambda i: (0, 0)),
        ],
        out_specs=[
            pl.BlockSpec((rt, lanes), lambda i: (i, 0)),
            pl.BlockSpec((1, lanes), lambda i: (0, 0)),
            pl.BlockSpec((rt, 128), lambda i: (i, 0)),
            pl.BlockSpec((rt, 128), lambda i: (i, 0)),
        ],
        out_shape=[
            jax.ShapeDtypeStruct((t, lanes), jnp.int32),
            jax.ShapeDtypeStruct((1, lanes), jnp.int32),
            jax.ShapeDtypeStruct((t, 128), jnp.float32),
            jax.ShapeDtypeStruct((t, 128), jnp.float32),
        ],
        scratch_shapes=[pltpu.VMEM((1, lanes), jnp.float32)],
    )(xf, wg_pad, bg_pad)
    return emat, counts, wrep0, wrep1


def _sc_dispatch(xf, slot0, slot1, wrep0, wrep1, np_rows):
    t, d = xf.shape
    info = plsc.get_sparse_core_info()
    nw = info.num_cores * info.num_subcores
    per_w = t // nw
    chunk = 32
    nch = per_w // chunk
    mesh = plsc.VectorSubcoreMesh(core_axis_name="c", subcore_axis_name="s")

    @functools.partial(
        pl.kernel, mesh=mesh,
        out_type=[
            jax.ShapeDtypeStruct((np_rows, d), jnp.float32),
            jax.ShapeDtypeStruct((np_rows, 128), jnp.float32),
        ],
        scratch_types=[
            pltpu.VMEM((2 * nch, chunk), jnp.int32),
            pltpu.VMEM((chunk, d), jnp.float32),
            pltpu.VMEM((chunk, d), jnp.float32),
            pltpu.VMEM((per_w, 128), jnp.float32),
            pltpu.VMEM((per_w, 128), jnp.float32),
            pltpu.SemaphoreType.DMA,
            pltpu.SemaphoreType.DMA,
            pltpu.SemaphoreType.DMA,
            pltpu.SemaphoreType.DMA,
            pltpu.SemaphoreType.DMA,
        ],
    )
    def dispatch_k(x_hbm, s0_hbm, s1_hbm, w0_hbm, w1_hbm,
                   xs_hbm, wr_hbm,
                   idx_v, rows0, rows1, w0_v, w1_v,
                   ld0, ld1, st0, st1, stw):
        wid = lax.axis_index("s") * info.num_cores + lax.axis_index("c")
        base = wid * per_w
        psl = pl.ds(base, per_w)
        pltpu.sync_copy(w0_hbm.at[psl], w0_v)
        pltpu.sync_copy(w1_hbm.at[psl], w1_v)
        for c in range(nch):
            csl2 = pl.ds(base + c * chunk, chunk)
            pltpu.sync_copy(s0_hbm.at[csl2], idx_v.at[c])
            pltpu.sync_copy(s1_hbm.at[csl2], idx_v.at[nch + c])

        rows = (rows0, rows1)
        lsem = (ld0, ld1)
        lcp = {}

        def fire(c):
            bi = c % 2
            lcp[c] = pltpu.async_copy(
                x_hbm.at[pl.ds(base + c * chunk, chunk)], rows[bi], lsem[bi])

        fire(0)
        scp = {}
        for c in range(nch):
            bi = c % 2
            lcp[c].wait()
            csl = pl.ds(c * chunk, chunk)
            scp[(c, 0)] = pltpu.async_copy(
                rows[bi], xs_hbm.at[idx_v.at[c]], st0)
            scp[(c, 1)] = pltpu.async_copy(
                rows[bi], xs_hbm.at[idx_v.at[nch + c]], st1)
            scp[(c, 2)] = pltpu.async_copy(
                w0_v.at[csl], wr_hbm.at[idx_v.at[c]], stw)
            scp[(c, 3)] = pltpu.async_copy(
                w1_v.at[csl], wr_hbm.at[idx_v.at[nch + c]], stw)
            if c + 1 < nch:
                if c >= 1:
                    for q in range(4):
                        scp[(c - 1, q)].wait()
                fire(c + 1)
        for c in (nch - 2, nch - 1):
            for q in range(4):
                scp[(c, q)].wait()

    return dispatch_k(xf, slot0, slot1, wrep0, wrep1)


def _ffn_body(te_ref, xs_ref, w1_ref, w2q0_ref, w2q1_ref, w2q2_ref,
              w2q3_ref, b1_ref, b2_ref, wrow_ref, out_ref, *, fh):
    pre = jnp.dot(xs_ref[...], w1_ref[0], preferred_element_type=jnp.float32)
    pre = pre + b1_ref[0]
    h = (0.5 * pre * (1.0 + jax.lax.erf(pre * 0.7071067811865476))
         ).astype(jnp.bfloat16)
    y = jnp.dot(h[:, :fh], w2q0_ref[0], preferred_element_type=jnp.float32)
    for q, wq in enumerate((w2q1_ref, w2q2_ref, w2q3_ref), start=1):
        y = y + jnp.dot(h[:, q * fh:(q + 1) * fh], wq[0],
                        preferred_element_type=jnp.float32)
    out_ref[...] = (y + b2_ref[0]) * wrow_ref[:, 0:1]


def _grouped_ffn(xs, W1, b1, W2, b2, tile_e, wrow):
    np_rows, d = xs.shape
    ne, _, dff = W1.shape
    nt = np_rows // TM
    b1r = b1.reshape(ne, 1, dff)
    b2r = b2.reshape(ne, 1, d)
    fh = dff // 4
    sb = pl.Buffered(buffer_count=1)
    return pl.pallas_call(
        functools.partial(_ffn_body, fh=fh),
        grid_spec=pltpu.PrefetchScalarGridSpec(
            num_scalar_prefetch=1,
            grid=(nt,),
            in_specs=[
                pl.BlockSpec((TM, d), lambda i, te: (i, 0)),
                pl.BlockSpec((1, d, dff), lambda i, te: (te[i], 0, 0)),
                pl.BlockSpec((1, fh, d), lambda i, te: (te[i], 0, 0)),
                pl.BlockSpec((1, fh, d), lambda i, te: (te[i], 1, 0),
                             pipeline_mode=sb),
                pl.BlockSpec((1, fh, d), lambda i, te: (te[i], 2, 0),
                             pipeline_mode=sb),
                pl.BlockSpec((1, fh, d), lambda i, te: (te[i], 3, 0),
                             pipeline_mode=sb),
                pl.BlockSpec((1, 1, dff), lambda i, te: (te[i], 0, 0)),
                pl.BlockSpec((1, 1, d), lambda i, te: (te[i], 0, 0)),
                pl.BlockSpec((TM, 128), lambda i, te: (i, 0)),
            ],
            out_specs=pl.BlockSpec((TM, d), lambda i, te: (i, 0)),
        ),
        out_shape=jax.ShapeDtypeStruct((np_rows, d), jnp.float32),
    )(tile_e, xs, W1, W2, W2, W2, W2, b1r, b2r, wrow)


def _sc_combine(yw, slot0, slot1):
    np_rows, d = yw.shape
    t = slot0.shape[0]
    info = plsc.get_sparse_core_info()
    nw = info.num_cores * info.num_subcores
    per_w = t // nw
    chunk = 16
    nch = per_w // chunk
    nvec = d // 16
    mesh = plsc.VectorSubcoreMesh(core_axis_name="c", subcore_axis_name="s")

    @functools.partial(
        pl.kernel, mesh=mesh,
        out_type=jax.ShapeDtypeStruct((t, d), jnp.float32),
        scratch_types=[
            pltpu.VMEM((per_w,), jnp.int32),
            pltpu.VMEM((per_w,), jnp.int32),
            pltpu.VMEM((chunk, d), jnp.float32),
            pltpu.VMEM((chunk, d), jnp.float32),
            pltpu.VMEM((chunk, d), jnp.float32),
            pltpu.VMEM((chunk, d), jnp.float32),
            pltpu.VMEM((chunk, d), jnp.float32),
            pltpu.VMEM((chunk, d), jnp.float32),
            pltpu.SemaphoreType.DMA,
            pltpu.SemaphoreType.DMA,
            pltpu.SemaphoreType.DMA,
            pltpu.SemaphoreType.DMA,
            pltpu.SemaphoreType.DMA,
            pltpu.SemaphoreType.DMA,
            pltpu.SemaphoreType.DMA,
            pltpu.SemaphoreType.DMA,
            pltpu.SemaphoreType.DMA,
        ],
    )
    def combine_k(y_hbm, s0_hbm, s1_hbm, out_hbm,
                  i0_v, i1_v,
                  a0, a1, a2, b0, b1, b2,
                  ga0, ga1, ga2, gb0, gb1, gb2, oa0, oa1, oa2):
        wid = lax.axis_index("s") * info.num_cores + lax.axis_index("c")
        base = wid * per_w
        pltpu.sync_copy(s0_hbm.at[pl.ds(base, per_w)], i0_v)
        pltpu.sync_copy(s1_hbm.at[pl.ds(base, per_w)], i1_v)
        bufa = (a0, a1, a2)
        bufb = (b0, b1, b2)
        gsa = (ga0, ga1, ga2)
        gsb = (gb0, gb1, gb2)
        osa = (oa0, oa1, oa2)
        cpa = {}
        cpb = {}
        ocp = {}

        def fire(j):
            bi = j % 3
            sl = pl.ds(j * chunk, chunk)
            cpa[j] = pltpu.async_copy(y_hbm.at[i0_v.at[sl]], bufa[bi], gsa[bi])
            cpb[j] = pltpu.async_copy(y_hbm.at[i1_v.at[sl]], bufb[bi], gsb[bi])

        fire(0)
        fire(1)
        for j in range(nch):
            bi = j % 3
            if j + 2 < nch:
                if j + 2 >= 3:
                    ocp[j - 1].wait()
                fire(j + 2)
            cpa[j].wait()
            cpb[j].wait()
            a = bufa[bi]
            b = bufb[bi]
            for r in range(chunk):
                def add_body(c, carry2, a=a, b=b, r=r):
                    sl2 = pl.ds(c * 16, 16)
                    a[r, sl2] = a[r, sl2] + b[r, sl2]
                    return carry2

                lax.fori_loop(0, nvec, add_body, 0)
            ocp[j] = pltpu.async_copy(
                bufa[bi], out_hbm.at[pl.ds(base + j * chunk, chunk)], osa[bi])
        ocp[nch - 3].wait()
        ocp[nch - 2].wait()
        ocp[nch - 1].wait()

    return combine_k(yw, slot0, slot1)


def kernel(x, Wg, bg, W1, b1, W2, b2):
    b, s, d = x.shape
    ne = W1.shape[0]
    t = b * s
    xf = x.reshape(t, d)
    emat, counts, wrep0, wrep1 = _router(xf, Wg, bg)

    counts8 = counts[0, :ne]
    padded = ((counts8 + TM - 1) // TM) * TM
    off_full = jnp.concatenate([jnp.zeros(1, jnp.int32), jnp.cumsum(padded)])
    off = off_full[:ne]
    np_rows = 2 * t + ne * TM
    nt = np_rows // TM
    starts = jnp.arange(nt, dtype=jnp.int32) * TM
    tile_e = jnp.minimum(
        jnp.sum((starts[:, None] >= off_full[None, 1:]).astype(jnp.int32),
                axis=1), ne - 1)

    e0a = emat[:, 0]
    e1a = emat[:, 1]
    r0a = emat[:, 2]
    r1a = emat[:, 3]

    slot0 = r0a
    slot1 = r1a
    for k in range(ne):
        slot0 = slot0 + jnp.where(e0a == k, off[k], 0)
        slot1 = slot1 + jnp.where(e1a == k, off[k], 0)

    xs, wrow = _sc_dispatch(xf, slot0, slot1, wrep0, wrep1, np_rows)
    yw = _grouped_ffn(xs, W1, b1, W2, b2, tile_e, wrow)
    outf = _sc_combine(yw, slot0, slot1)
    return outf.reshape(b, s, d)
